# Initial kernel scaffold; baseline (speedup 1.0000x reference)
import jax
import jax.numpy as jnp
from jax.experimental import pallas as pl


def kernel(node_feat, edge_index, edge_attr, W1, b1, W2, b2, W3, b3, W4, b4):
    raise NotImplementedError("write your pallas kernel here")


# speedup vs baseline: 3.1388x; 3.1388x over previous
import dataclasses
import functools

import jax
import jax.numpy as jnp
from jax import lax
from jax.experimental import pallas as pl
from jax.experimental.pallas import tpu as pltpu
from jax.experimental.pallas import tpu_sc as plsc

N_NODES = 10000
N_EDGES = 320000
D_IN = 128
D_H = 128
NC = 2
NS = 16
LANES = 16
C = 64
N_CHUNKS = N_EDGES // C
FULL_BLOCKS = N_NODES // C
TAIL_ROWS = N_NODES - FULL_BLOCKS * C
H_ROWS = 80
HIGH = jax.lax.Precision.HIGHEST

_sc_mesh = plsc.VectorSubcoreMesh(
    core_axis_name="c", subcore_axis_name="s", num_cores=NC, num_subcores=NS)

_sc_params = pltpu.CompilerParams()
if "needs_layout_passes" in pltpu.CompilerParams.__dataclass_fields__:
    _sc_params = dataclasses.replace(_sc_params, needs_layout_passes=False)


@functools.partial(
    pl.kernel,
    out_type=(jax.ShapeDtypeStruct((NC, N_NODES, D_H), jnp.float32),
              jax.ShapeDtypeStruct((NC, H_ROWS, D_H), jnp.float32)),
    mesh=_sc_mesh,
    compiler_params=_sc_params,
    scratch_types=[
        pltpu.VMEM((C,), jnp.int32),
        pltpu.VMEM((C,), jnp.int32),
        pltpu.VMEM((C, D_H), jnp.float32),
        pltpu.VMEM((C, D_H), jnp.float32),
        pltpu.VMEM((C, D_H), jnp.float32),
        pltpu.VMEM((H_ROWS, D_H), jnp.float32),
        pltpu.VMEM((H_ROWS,), jnp.int32),
        pltpu.SemaphoreType.DMA,
        pltpu.VMEM_SHARED((N_NODES, D_H), jnp.float32),
        pltpu.VMEM_SHARED((H_ROWS, D_H), jnp.float32),
    ],
)
def _sc_segment_kernel(p_hbm, e_hbm, col_hbm, row_hbm, s_out, cnt_out,
                       colbuf, rowbuf, pbuf, ebuf, xbuf, histbuf, iotabuf,
                       sem, s_shared, cnt_shared):
    cid = lax.axis_index("c")
    sid = lax.axis_index("s")
    wid = sid * NC + cid
    zeros16 = jnp.zeros((LANES,), jnp.float32)
    ones16 = jnp.ones((LANES,), jnp.float32)
    lane16 = lax.iota(jnp.int32, 16)

    @pl.loop(0, C)
    def _(r):
        for k in range(D_H // LANES):
            xbuf[r, pl.ds(k * LANES, LANES)] = zeros16

    @pl.loop(0, H_ROWS)
    def _(r):
        for k in range(D_H // LANES):
            histbuf[r, pl.ds(k * LANES, LANES)] = zeros16

    for k in range(H_ROWS // LANES):
        iotabuf[pl.ds(k * LANES, LANES)] = lane16 + (k * LANES)

    @pl.loop(sid, FULL_BLOCKS, step=NS)
    def _(b):
        pltpu.sync_copy(xbuf, s_shared.at[pl.ds(b * C, C)])

    @pl.when(sid == 0)
    def _():
        pltpu.sync_copy(xbuf.at[pl.ds(0, TAIL_ROWS)],
                        s_shared.at[pl.ds(FULL_BLOCKS * C, TAIL_ROWS)])

    @pl.when(sid == 1)
    def _():
        pltpu.sync_copy(xbuf.at[pl.ds(0, H_ROWS)], cnt_shared)

    plsc.subcore_barrier()

    @pl.loop(wid, N_CHUNKS, step=NC * NS)
    def _(chunk):
        base = chunk * C
        pltpu.sync_copy(col_hbm.at[pl.ds(base, C)], colbuf)
        gather = pltpu.async_copy(p_hbm.at[colbuf], pbuf, sem)
        pltpu.sync_copy(row_hbm.at[pl.ds(base, C)], rowbuf)
        pltpu.sync_copy(e_hbm.at[pl.ds(base, C)], ebuf)
        gather.wait()

        @pl.loop(0, C)
        def _(r):
            for k in range(D_H // LANES):
                sl = pl.ds(k * LANES, LANES)
                xbuf[r, sl] = jnp.maximum(pbuf[r, sl] + ebuf[r, sl], 0.0)

        for k in range(C // LANES):
            rv = rowbuf[pl.ds(k * LANES, LANES)]
            plsc.addupdate_scatter(
                histbuf, [lax.shift_right_logical(rv, 7),
                          lax.bitwise_and(rv, 127)], ones16)

        pltpu.sync_copy(xbuf, s_shared.at[rowbuf], add=True)

    pltpu.sync_copy(histbuf, cnt_shared.at[iotabuf], add=True)

    plsc.subcore_barrier()

    @pl.loop(sid, FULL_BLOCKS, step=NS)
    def _(b):
        pltpu.sync_copy(s_shared.at[pl.ds(b * C, C)],
                        s_out.at[cid].at[pl.ds(b * C, C)])

    @pl.when(sid == 0)
    def _():
        pltpu.sync_copy(s_shared.at[pl.ds(FULL_BLOCKS * C, TAIL_ROWS)],
                        s_out.at[cid].at[pl.ds(FULL_BLOCKS * C, TAIL_ROWS)])

    @pl.when(sid == 1)
    def _():
        pltpu.sync_copy(cnt_shared, cnt_out.at[cid])


def _node_proj_body(nf_ref, w_ref, b_ref, out_ref):
    out_ref[...] = lax.dot_general(
        nf_ref[...], w_ref[...], (((1,), (0,)), ((), ())),
        preferred_element_type=jnp.float32, precision=HIGH) + b_ref[...]


def _edge_proj_body(ea_ref, w_ref, out_ref):
    out_ref[...] = lax.dot_general(
        ea_ref[...], w_ref[...], (((1,), (0,)), ((), ())),
        preferred_element_type=jnp.float32, precision=HIGH)


def _final_body(p0_ref, p1_ref, c0_ref, c1_ref, nf_ref, w2_ref, b2_ref,
                w3n_ref, w3m_ref, b3_ref, w4_ref, b4_ref, out_ref):
    s = p0_ref[...] + p1_ref[...]
    cnt = c0_ref[...] + c1_ref[...]
    sum_t = lax.dot_general(s, w2_ref[...], (((1,), (0,)), ((), ())),
                            preferred_element_type=jnp.float32,
                            precision=HIGH) + cnt * b2_ref[...]
    agg = sum_t / jnp.maximum(cnt, 1.0)
    u = lax.dot_general(nf_ref[...], w3n_ref[...], (((1,), (0,)), ((), ())),
                        preferred_element_type=jnp.float32, precision=HIGH)
    u = u + lax.dot_general(agg, w3m_ref[...], (((1,), (0,)), ((), ())),
                            preferred_element_type=jnp.float32,
                            precision=HIGH) + b3_ref[...]
    u = jnp.maximum(u, 0.0)
    out_ref[...] = lax.dot_general(
        u, w4_ref[...], (((1,), (0,)), ((), ())),
        preferred_element_type=jnp.float32, precision=HIGH) + b4_ref[...]


def kernel(node_feat, edge_index, edge_attr, W1, b1, W2, b2, W3, b3, W4, b4):
    row = edge_index[0]
    col = edge_index[1]
    w1n, w1e = W1[:D_IN], W1[D_IN:]
    w3n, w3m = W3[:D_IN], W3[D_IN:]

    p = pl.pallas_call(
        _node_proj_body,
        out_shape=jax.ShapeDtypeStruct((N_NODES, D_H), jnp.float32),
    )(node_feat, w1n, b1.reshape(1, D_H))

    eb = 8000
    e = pl.pallas_call(
        _edge_proj_body,
        grid=(N_EDGES // eb,),
        in_specs=[pl.BlockSpec((eb, 16), lambda i: (i, 0)),
                  pl.BlockSpec((16, D_H), lambda i: (0, 0))],
        out_specs=pl.BlockSpec((eb, D_H), lambda i: (i, 0)),
        out_shape=jax.ShapeDtypeStruct((N_EDGES, D_H), jnp.float32),
    )(edge_attr, w1e)

    partials, counts = _sc_segment_kernel(p, e, col, row)
    cnt0 = counts[0].reshape(H_ROWS * D_H, 1)[:N_NODES]
    cnt1 = counts[1].reshape(H_ROWS * D_H, 1)[:N_NODES]

    nb = 1000
    out = pl.pallas_call(
        _final_body,
        grid=(N_NODES // nb,),
        in_specs=[pl.BlockSpec((nb, D_H), lambda i: (i, 0)),
                  pl.BlockSpec((nb, D_H), lambda i: (i, 0)),
                  pl.BlockSpec((nb, 1), lambda i: (i, 0)),
                  pl.BlockSpec((nb, 1), lambda i: (i, 0)),
                  pl.BlockSpec((nb, D_IN), lambda i: (i, 0)),
                  pl.BlockSpec((D_H, D_H), lambda i: (0, 0)),
                  pl.BlockSpec((1, D_H), lambda i: (0, 0)),
                  pl.BlockSpec((D_IN, D_H), lambda i: (0, 0)),
                  pl.BlockSpec((D_H, D_H), lambda i: (0, 0)),
                  pl.BlockSpec((1, D_H), lambda i: (0, 0)),
                  pl.BlockSpec((D_H, D_H), lambda i: (0, 0)),
                  pl.BlockSpec((1, D_H), lambda i: (0, 0))],
        out_specs=pl.BlockSpec((nb, D_H), lambda i: (i, 0)),
        out_shape=jax.ShapeDtypeStruct((N_NODES, D_H), jnp.float32),
    )(partials[0], partials[1], cnt0, cnt1, node_feat, W2, b2.reshape(1, D_H),
      w3n, w3m, b3.reshape(1, D_H), W4, b4.reshape(1, D_H))

    return out
